# baseline (speedup 1.0000x reference)
import jax
import jax.numpy as jnp
from jax.experimental import pallas as pl
from jax.experimental.pallas import tpu as pltpu

_N = 20000
_NCLS = 80
_K = 2000
_CONF_BITS = 0x3E800000
_HI_BITS = 0x7F7FFFFF
_GRID = 8
_N_PAD = 20480
_CHUNK = _N_PAD // _GRID


def _body(x_ref, o_ref, s_scr, c_scr):
    pid = pl.program_id(0)
    x = x_ref[...]
    scores = jnp.max(x[0:_NCLS, :], axis=0, keepdims=True)
    bsum = jnp.sum(x[_NCLS:_NCLS + 4, :], axis=0, keepdims=True)
    s_scr[pl.ds(pid, 1), :] = scores
    c_scr[pl.ds(pid, 1), :] = scores + bsum

    @pl.when(pid == _GRID - 1)
    def _():
        s = s_scr[...]
        si = jax.lax.bitcast_convert_type(s, jnp.int32)
        c = c_scr[...]
        kk = jnp.int32(_K)

        def bs_body(_, carry):
            lo, hi = carry
            mid = lo + (hi - lo + 1) // 2
            cnt = jnp.sum((si >= mid).astype(jnp.int32))
            ok = cnt >= kk
            return jnp.where(ok, mid, lo), jnp.where(ok, hi, mid - 1)

        t, _unused = jax.lax.fori_loop(
            0, 32, bs_body,
            (jnp.int32(_CONF_BITS), jnp.int32(_HI_BITS)))

        gt = si > t
        eq = si == t
        cnt_gt = jnp.sum(gt.astype(jnp.int32))
        cnt_eq = jnp.sum(eq.astype(jnp.int32))
        need = jnp.maximum(jnp.minimum(kk - cnt_gt, cnt_eq), 0)

        r = jax.lax.broadcasted_iota(jnp.int32, (_GRID, _CHUNK), 0)
        col = jax.lax.broadcasted_iota(jnp.int32, (_GRID, _CHUNK), 1)
        idx = r * _CHUNK + col

        def j_body(_, carry):
            lo2, hi2 = carry
            mid = (lo2 + hi2) // 2
            cnt = jnp.sum((eq & (idx < mid)).astype(jnp.int32))
            ok = cnt >= need
            return jnp.where(ok, lo2, mid + 1), jnp.where(ok, mid, hi2)

        _unused2, j = jax.lax.fori_loop(
            0, 15, j_body, (jnp.int32(0), jnp.int32(_N)))

        sel = gt | (eq & (idx < j))
        o_ref[...] = jnp.reshape(jnp.sum(jnp.where(sel, c, 0.0)), (1, 1))


def kernel(data):
    x = data[0]
    xt = jnp.concatenate([x[:, 4:4 + _NCLS].T, x[:, 0:4].T], axis=0)
    xt = jnp.pad(xt, ((0, 0), (0, _N_PAD - _N)),
                 constant_values=-jnp.inf)
    out = pl.pallas_call(
        _body,
        grid=(_GRID,),
        in_specs=[pl.BlockSpec((_NCLS + 4, _CHUNK), lambda i: (0, i))],
        out_specs=pl.BlockSpec((1, 1), lambda i: (0, 0)),
        out_shape=jax.ShapeDtypeStruct((1, 1), jnp.float32),
        scratch_shapes=[
            pltpu.VMEM((_GRID, _CHUNK), jnp.float32),
            pltpu.VMEM((_GRID, _CHUNK), jnp.float32),
        ],
    )(xt)
    return jnp.reshape(out, ())


# speedup vs baseline: 1.0364x; 1.0364x over previous
import jax
import jax.numpy as jnp
from jax.experimental import pallas as pl
from jax.experimental.pallas import tpu as pltpu

_N = 20000
_NCLS = 80
_K = 2000
_CONF_BITS = 0x3E800000
_HI_BITS = 0x7F7FFFFF
_GRID = 8
_RPB = 2560
_ROWS = _RPB // 128
_SROWS = _GRID * _ROWS


def _body(x_ref, o_ref, s_scr, c_scr):
    pid = pl.program_id(0)
    x = x_ref[...]
    scores = jnp.max(x[:, 4:4 + _NCLS], axis=1)
    bsum = jnp.sum(x[:, 0:4], axis=1)
    s2 = jnp.reshape(scores, (_ROWS, 128))
    c2 = jnp.reshape(scores + bsum, (_ROWS, 128))
    r = jax.lax.broadcasted_iota(jnp.int32, (_ROWS, 128), 0)
    l = jax.lax.broadcasted_iota(jnp.int32, (_ROWS, 128), 1)
    valid = (pid * _RPB + r * 128 + l) < _N
    s2 = jnp.where(valid, s2, -jnp.inf)
    s_scr[pl.ds(pid * _ROWS, _ROWS), :] = s2
    c_scr[pl.ds(pid * _ROWS, _ROWS), :] = c2

    @pl.when(pid == _GRID - 1)
    def _():
        s = s_scr[...]
        si = jax.lax.bitcast_convert_type(s, jnp.int32)
        c = c_scr[...]
        kk = jnp.int32(_K)

        def bs_body(_, carry):
            lo, hi = carry
            mid = lo + (hi - lo + 1) // 2
            cnt = jnp.sum((si >= mid).astype(jnp.int32))
            ok = cnt >= kk
            return jnp.where(ok, mid, lo), jnp.where(ok, hi, mid - 1)

        t, _unused = jax.lax.fori_loop(
            0, 32, bs_body,
            (jnp.int32(_CONF_BITS), jnp.int32(_HI_BITS)))

        gt = si > t
        eq = si == t
        cnt_gt = jnp.sum(gt.astype(jnp.int32))
        cnt_eq = jnp.sum(eq.astype(jnp.int32))
        need = jnp.maximum(jnp.minimum(kk - cnt_gt, cnt_eq), 0)

        r2 = jax.lax.broadcasted_iota(jnp.int32, (_SROWS, 128), 0)
        l2 = jax.lax.broadcasted_iota(jnp.int32, (_SROWS, 128), 1)
        idx = r2 * 128 + l2

        def j_body(_, carry):
            lo2, hi2 = carry
            mid = (lo2 + hi2) // 2
            cnt = jnp.sum((eq & (idx < mid)).astype(jnp.int32))
            ok = cnt >= need
            return jnp.where(ok, lo2, mid + 1), jnp.where(ok, mid, hi2)

        _unused2, j = jax.lax.fori_loop(
            0, 15, j_body, (jnp.int32(0), jnp.int32(_N)))

        sel = gt | (eq & (idx < j))
        o_ref[...] = jnp.reshape(jnp.sum(jnp.where(sel, c, 0.0)), (1, 1))


def kernel(data):
    x = data[0]
    out = pl.pallas_call(
        _body,
        grid=(_GRID,),
        in_specs=[pl.BlockSpec((_RPB, 84), lambda i: (i, 0))],
        out_specs=pl.BlockSpec((1, 1), lambda i: (0, 0)),
        out_shape=jax.ShapeDtypeStruct((1, 1), jnp.float32),
        scratch_shapes=[
            pltpu.VMEM((_SROWS, 128), jnp.float32),
            pltpu.VMEM((_SROWS, 128), jnp.float32),
        ],
    )(x)
    return jnp.reshape(out, ())


# speedup vs baseline: 1.7595x; 1.6977x over previous
import jax
import jax.numpy as jnp
from jax.experimental import pallas as pl
from jax.experimental.pallas import tpu as pltpu

_N = 20000
_NCLS = 80
_K = 2000
_CONF_BITS = 0x3E800000
_HI_BITS = 0x7F7FFFFF
_GRID = 8
_RPB = 2560
_ROWS = _RPB // 128
_SROWS = _GRID * _ROWS


def _body(x_ref, o_ref, s_scr, c_scr):
    pid = pl.program_id(0)
    x = x_ref[0]
    scores = jnp.max(x[:, 4:4 + _NCLS], axis=1)
    bsum = jnp.sum(x[:, 0:4], axis=1)
    s2 = jnp.reshape(scores, (_ROWS, 128))
    c2 = jnp.reshape(scores + bsum, (_ROWS, 128))
    r = jax.lax.broadcasted_iota(jnp.int32, (_ROWS, 128), 0)
    l = jax.lax.broadcasted_iota(jnp.int32, (_ROWS, 128), 1)
    valid = (pid * _RPB + r * 128 + l) < _N
    s2 = jnp.where(valid, s2, -jnp.inf)
    s_scr[pl.ds(pid * _ROWS, _ROWS), :] = s2
    c_scr[pl.ds(pid * _ROWS, _ROWS), :] = c2

    @pl.when(pid == _GRID - 1)
    def _():
        s = s_scr[...]
        si = jax.lax.bitcast_convert_type(s, jnp.int32)
        c = c_scr[...]
        kk = jnp.int32(_K)

        def bs_body(_, carry):
            lo, hi = carry
            mid = lo + (hi - lo + 1) // 2
            cnt = jnp.sum((si >= mid).astype(jnp.int32))
            ok = cnt >= kk
            return jnp.where(ok, mid, lo), jnp.where(ok, hi, mid - 1)

        t, _unused = jax.lax.fori_loop(
            0, 32, bs_body,
            (jnp.int32(_CONF_BITS), jnp.int32(_HI_BITS)))

        gt = si > t
        eq = si == t
        cnt_gt = jnp.sum(gt.astype(jnp.int32))
        cnt_eq = jnp.sum(eq.astype(jnp.int32))
        need = jnp.maximum(jnp.minimum(kk - cnt_gt, cnt_eq), 0)

        r2 = jax.lax.broadcasted_iota(jnp.int32, (_SROWS, 128), 0)
        l2 = jax.lax.broadcasted_iota(jnp.int32, (_SROWS, 128), 1)
        idx = r2 * 128 + l2

        def j_body(_, carry):
            lo2, hi2 = carry
            mid = (lo2 + hi2) // 2
            cnt = jnp.sum((eq & (idx < mid)).astype(jnp.int32))
            ok = cnt >= need
            return jnp.where(ok, lo2, mid + 1), jnp.where(ok, mid, hi2)

        _unused2, j = jax.lax.fori_loop(
            0, 15, j_body, (jnp.int32(0), jnp.int32(_N)))

        sel = gt | (eq & (idx < j))
        o_ref[...] = jnp.reshape(jnp.sum(jnp.where(sel, c, 0.0)), (1, 1))


def kernel(data):
    out = pl.pallas_call(
        _body,
        grid=(_GRID,),
        in_specs=[pl.BlockSpec((1, _RPB, 84), lambda i: (0, i, 0))],
        out_specs=pl.BlockSpec((1, 1), lambda i: (0, 0)),
        out_shape=jax.ShapeDtypeStruct((1, 1), jnp.float32),
        scratch_shapes=[
            pltpu.VMEM((_SROWS, 128), jnp.float32),
            pltpu.VMEM((_SROWS, 128), jnp.float32),
        ],
    )(data)
    return jnp.reshape(out, ())


# speedup vs baseline: 2.1037x; 1.1956x over previous
import jax
import jax.numpy as jnp
from jax.experimental import pallas as pl
from jax.experimental.pallas import tpu as pltpu

_N = 20000
_NCLS = 80
_K = 2000
_CONF_BITS = 0x3E800000
_HI_BITS = 0x7F7FFFFF
_GRID = 4
_RPB = 5120
_ROWS = _RPB // 128
_SROWS = _GRID * _ROWS


def _body(x_ref, o_ref, s_scr, c_scr):
    pid = pl.program_id(0)
    x = x_ref[0]
    scores = jnp.max(x[:, 4:4 + _NCLS], axis=1)
    bsum = jnp.sum(x[:, 0:4], axis=1)
    s2 = jnp.reshape(scores, (_ROWS, 128))
    c2 = jnp.reshape(scores + bsum, (_ROWS, 128))
    r = jax.lax.broadcasted_iota(jnp.int32, (_ROWS, 128), 0)
    l = jax.lax.broadcasted_iota(jnp.int32, (_ROWS, 128), 1)
    valid = (pid * _RPB + r * 128 + l) < _N
    s2 = jnp.where(valid, s2, -jnp.inf)
    s_scr[pl.ds(pid * _ROWS, _ROWS), :] = s2
    c_scr[pl.ds(pid * _ROWS, _ROWS), :] = c2

    @pl.when(pid == _GRID - 1)
    def _():
        s = s_scr[...]
        si = jax.lax.bitcast_convert_type(s, jnp.int32)
        c = c_scr[...]
        kk = jnp.int32(_K)

        u = jnp.maximum(si, jnp.int32(_CONF_BITS)) - jnp.int32(_CONF_BITS)
        base = jnp.int32(0)
        for shift in (27, 23, 19, 15, 11, 7, 3, 0):
            bc = jnp.clip((u - base) >> shift, 0, 15)
            jstar = jnp.int32(0)
            for jj in range(1, 16):
                cnt = jnp.sum((bc >= jj).astype(jnp.int32))
                jstar = jstar + (cnt >= kk).astype(jnp.int32)
            base = base + (jstar << shift)
        t = jnp.int32(_CONF_BITS) + base

        gt = si > t
        eq = si == t
        cnt_gt = jnp.sum(gt.astype(jnp.int32))
        cnt_eq = jnp.sum(eq.astype(jnp.int32))
        need = jnp.maximum(jnp.minimum(kk - cnt_gt, cnt_eq), 0)

        r2 = jax.lax.broadcasted_iota(jnp.int32, (_SROWS, 128), 0)
        l2 = jax.lax.broadcasted_iota(jnp.int32, (_SROWS, 128), 1)
        idx = r2 * 128 + l2

        def tie_j():
            idxv = jnp.where(eq, idx, jnp.int32(1 << 30))
            jb = jnp.int32(0)
            for shift in (11, 7, 3, 0):
                g = jnp.clip((idxv - jb) >> shift, -1, 16)
                nsat = jnp.int32(0)
                for jj in range(1, 17):
                    cnt = jnp.sum((g < jj).astype(jnp.int32))
                    nsat = nsat + (cnt >= need).astype(jnp.int32)
                jb = jb + ((jnp.int32(17) - nsat - 1) << shift)
            return jb + 1

        j = jax.lax.cond(need == cnt_eq,
                         lambda: jnp.int32(_N), tie_j)
        j = jnp.where(need > 0, j, 0)

        sel = gt | (eq & (idx < j))
        o_ref[...] = jnp.reshape(jnp.sum(jnp.where(sel, c, 0.0)), (1, 1))


def kernel(data):
    out = pl.pallas_call(
        _body,
        grid=(_GRID,),
        in_specs=[pl.BlockSpec((1, _RPB, 84), lambda i: (0, i, 0))],
        out_specs=pl.BlockSpec((1, 1), lambda i: (0, 0)),
        out_shape=jax.ShapeDtypeStruct((1, 1), jnp.float32),
        scratch_shapes=[
            pltpu.VMEM((_SROWS, 128), jnp.float32),
            pltpu.VMEM((_SROWS, 128), jnp.float32),
        ],
    )(data)
    return jnp.reshape(out, ())
